# Initial kernel scaffold; baseline (speedup 1.0000x reference)
import jax
import jax.numpy as jnp
from jax.experimental import pallas as pl


def kernel(x, edge_index, W1, b1, W2, b2):
    raise NotImplementedError("write your pallas kernel here")


# speedup vs baseline: 103.6280x; 103.6280x over previous
import jax
import jax.numpy as jnp
from jax import lax
from jax.experimental import pallas as pl
from jax.experimental.pallas import tpu as pltpu
from jax.experimental.pallas import tpu_sc as plsc

N_NODES = 100000
N_EDGES = 1600000
NPAD = 102400
NC, NS = 2, 16
NW = NC * NS
PER_W = N_EDGES // NW
CHUNK = 2000
NCHUNK = PER_W // CHUNK
SLICE = NPAD // NS

_f32 = jnp.float32


def _mesh():
    return plsc.VectorSubcoreMesh(
        core_axis_name="c", subcore_axis_name="s", num_cores=NC, num_subcores=NS
    )


def _deg_body(dst_hbm, zeros_hbm, ones_hbm, out_hbm, acc, dstv, onesv):
    c = lax.axis_index("c")
    s = lax.axis_index("s")
    w = s * NC + c
    sl = pl.ds(s * SLICE, SLICE)
    pltpu.sync_copy(zeros_hbm, acc.at[sl])
    pltpu.sync_copy(ones_hbm, onesv)
    plsc.subcore_barrier()

    def body(k, carry):
        base = w * PER_W + k * CHUNK
        pltpu.sync_copy(dst_hbm.at[pl.ds(base, CHUNK)], dstv)
        pltpu.sync_copy(onesv, acc.at[dstv], add=True)
        return carry

    lax.fori_loop(0, NCHUNK, body, 0)
    plsc.subcore_barrier()
    pltpu.sync_copy(acc.at[sl], out_hbm.at[c, sl])


def _l1_body(src_hbm, dst_hbm, t0_hbm, t1_hbm, zeros_hbm, out_hbm,
             acc0, acc1, srcv, dstv, v0, v1, sem):
    c = lax.axis_index("c")
    s = lax.axis_index("s")
    w = s * NC + c
    sl = pl.ds(s * SLICE, SLICE)
    pltpu.sync_copy(zeros_hbm, acc0.at[sl])
    pltpu.sync_copy(zeros_hbm, acc1.at[sl])
    plsc.subcore_barrier()

    def body(k, carry):
        base = w * PER_W + k * CHUNK
        pltpu.sync_copy(src_hbm.at[pl.ds(base, CHUNK)], srcv)
        pltpu.sync_copy(dst_hbm.at[pl.ds(base, CHUNK)], dstv)
        pltpu.async_copy(t0_hbm.at[srcv], v0, sem).wait()
        pltpu.async_copy(t1_hbm.at[srcv], v1, sem).wait()
        pltpu.sync_copy(v0, acc0.at[dstv], add=True)
        pltpu.sync_copy(v1, acc1.at[dstv], add=True)
        return carry

    lax.fori_loop(0, NCHUNK, body, 0)
    plsc.subcore_barrier()
    pltpu.sync_copy(acc0.at[sl], out_hbm.at[c, 0, sl])
    pltpu.sync_copy(acc1.at[sl], out_hbm.at[c, 1, sl])


def _l2_body(src_hbm, dst_hbm, t0_hbm, zeros_hbm, out_hbm,
             acc0, srcv, dstv, v0, sem):
    c = lax.axis_index("c")
    s = lax.axis_index("s")
    w = s * NC + c
    sl = pl.ds(s * SLICE, SLICE)
    pltpu.sync_copy(zeros_hbm, acc0.at[sl])
    plsc.subcore_barrier()

    def body(k, carry):
        base = w * PER_W + k * CHUNK
        pltpu.sync_copy(src_hbm.at[pl.ds(base, CHUNK)], srcv)
        pltpu.sync_copy(dst_hbm.at[pl.ds(base, CHUNK)], dstv)
        pltpu.async_copy(t0_hbm.at[srcv], v0, sem).wait()
        pltpu.sync_copy(v0, acc0.at[dstv], add=True)
        return carry

    lax.fori_loop(0, NCHUNK, body, 0)
    plsc.subcore_barrier()
    pltpu.sync_copy(acc0.at[sl], out_hbm.at[c, sl])


_deg_call = pl.kernel(
    _deg_body,
    out_type=jax.ShapeDtypeStruct((NC, NPAD), _f32),
    mesh=_mesh(),
    scratch_types=[
        pltpu.VMEM_SHARED((NPAD,), _f32),
        pltpu.VMEM((CHUNK,), jnp.int32),
        pltpu.VMEM((CHUNK,), _f32),
    ],
)

_l1_call = pl.kernel(
    _l1_body,
    out_type=jax.ShapeDtypeStruct((NC, 2, NPAD), _f32),
    mesh=_mesh(),
    scratch_types=[
        pltpu.VMEM_SHARED((NPAD,), _f32),
        pltpu.VMEM_SHARED((NPAD,), _f32),
        pltpu.VMEM((CHUNK,), jnp.int32),
        pltpu.VMEM((CHUNK,), jnp.int32),
        pltpu.VMEM((CHUNK,), _f32),
        pltpu.VMEM((CHUNK,), _f32),
        pltpu.SemaphoreType.DMA,
    ],
)

_l2_call = pl.kernel(
    _l2_body,
    out_type=jax.ShapeDtypeStruct((NC, NPAD), _f32),
    mesh=_mesh(),
    scratch_types=[
        pltpu.VMEM_SHARED((NPAD,), _f32),
        pltpu.VMEM((CHUNK,), jnp.int32),
        pltpu.VMEM((CHUNK,), jnp.int32),
        pltpu.VMEM((CHUNK,), _f32),
        pltpu.SemaphoreType.DMA,
    ],
)


def _tc1_body(degp, x0, x1, dinv_o, y10_o, y11_o):
    deg = degp[0:1, :] + degp[1:2, :] + 1.0
    dinv = lax.rsqrt(deg)
    dinv_o[...] = dinv
    y10_o[...] = dinv * x0[...]
    y11_o[...] = dinv * x1[...]


def _tc2_body(a1p, x0, x1, dinv, w1t, b1, w2t, z_o, y2_o):
    dv = dinv[...]
    d2 = dv * dv
    ap = a1p[0] + a1p[1]
    xx = jnp.concatenate([x0[...], x1[...]], axis=0)
    ax = dv * ap + d2 * xx
    h = jnp.dot(w1t[...], ax, preferred_element_type=_f32) + b1[...]
    h = jnp.maximum(h, 0.0)
    z = jnp.dot(w2t[...], h, preferred_element_type=_f32)
    z_o[...] = z
    y2_o[...] = dv * z


def _tc3_body(a2p, z, dinv, b2, out_o):
    dv = dinv[...]
    out_o[...] = dv * (a2p[0:1, :] + a2p[1:2, :] + dv * z[...]) + b2[...]


_tc1_call = pl.pallas_call(
    _tc1_body,
    out_shape=(
        jax.ShapeDtypeStruct((1, NPAD), _f32),
        jax.ShapeDtypeStruct((1, NPAD), _f32),
        jax.ShapeDtypeStruct((1, NPAD), _f32),
    ),
)

_tc2_call = pl.pallas_call(
    _tc2_body,
    out_shape=(
        jax.ShapeDtypeStruct((1, NPAD), _f32),
        jax.ShapeDtypeStruct((1, NPAD), _f32),
    ),
)

_tc3_call = pl.pallas_call(
    _tc3_body,
    out_shape=jax.ShapeDtypeStruct((1, NPAD), _f32),
)


def kernel(x, edge_index, W1, b1, W2, b2):
    src = edge_index[0].astype(jnp.int32)
    dst = edge_index[1].astype(jnp.int32)
    pad = NPAD - N_NODES
    x0 = jnp.pad(x[:, 0], (0, pad)).reshape(1, NPAD)
    x1 = jnp.pad(x[:, 1], (0, pad)).reshape(1, NPAD)
    zeros_h = jnp.zeros((SLICE,), _f32)
    ones_h = jnp.ones((CHUNK,), _f32)
    w1t = W1.T
    w2t = W2.T
    b1c = b1.reshape(64, 1)
    b2c = b2.reshape(1, 1)

    degp = _deg_call(dst, zeros_h, ones_h)
    dinv, y10, y11 = _tc1_call(degp, x0, x1)
    a1p = _l1_call(src, dst, y10.reshape(NPAD), y11.reshape(NPAD), zeros_h)
    z, y2 = _tc2_call(a1p, x0, x1, dinv, w1t, b1c, w2t)
    a2p = _l2_call(src, dst, y2.reshape(NPAD), zeros_h)
    out = _tc3_call(a2p, z, dinv, b2c)
    return out.reshape(NPAD)[:N_NODES]


# speedup vs baseline: 142.9998x; 1.3799x over previous
import jax
import jax.numpy as jnp
from jax import lax
from jax.experimental import pallas as pl
from jax.experimental.pallas import tpu as pltpu
from jax.experimental.pallas import tpu_sc as plsc

N_NODES = 100000
N_EDGES = 1600000
NPAD = 102400
NC, NS = 2, 16
NW = NC * NS
PER_W = N_EDGES // NW
CHUNK = 2000
NCHUNK = PER_W // CHUNK
SLICE = NPAD // NS

_f32 = jnp.float32


def _mesh():
    return plsc.VectorSubcoreMesh(
        core_axis_name="c", subcore_axis_name="s", num_cores=NC, num_subcores=NS
    )


def _deg_body(dst_hbm, zeros_hbm, ones_hbm, out_hbm, acc,
              dstv0, dstv1, onesv, sem_i0, sem_i1, sem_s0, sem_s1):
    c = lax.axis_index("c")
    s = lax.axis_index("s")
    w = s * NC + c
    sl = pl.ds(s * SLICE, SLICE)
    pltpu.sync_copy(zeros_hbm, acc.at[sl])
    pltpu.sync_copy(ones_hbm, onesv)
    plsc.subcore_barrier()

    dstv = (dstv0, dstv1)
    sem_i = (sem_i0, sem_i1)
    sem_s = (sem_s0, sem_s1)
    i_d = [None] * NCHUNK
    s_d = [None] * NCHUNK
    for k in range(NCHUNK):
        b = k % 2
        if k >= 2:
            s_d[k - 2].wait()
        base = w * PER_W + k * CHUNK
        i_d[k] = pltpu.async_copy(dst_hbm.at[pl.ds(base, CHUNK)], dstv[b], sem_i[b])
        i_d[k].wait()
        s_d[k] = pltpu.async_copy(onesv, acc.at[dstv[b]], sem_s[b], add=True)
    s_d[NCHUNK - 2].wait()
    s_d[NCHUNK - 1].wait()
    plsc.subcore_barrier()
    pltpu.sync_copy(acc.at[sl], out_hbm.at[c, sl])


def _l1_body(src_hbm, dst_hbm, t0_hbm, t1_hbm, zeros_hbm, out_hbm,
             acc0, acc1, srcv0, srcv1, dstv0, dstv1,
             v00, v01, v10, v11,
             sem_i, sem_g0, sem_g1, sem_s0, sem_s1):
    c = lax.axis_index("c")
    s = lax.axis_index("s")
    w = s * NC + c
    sl = pl.ds(s * SLICE, SLICE)
    pltpu.sync_copy(zeros_hbm, acc0.at[sl])
    pltpu.sync_copy(zeros_hbm, acc1.at[sl])
    plsc.subcore_barrier()

    srcv = (srcv0, srcv1)
    dstv = (dstv0, dstv1)
    v0 = (v00, v01)
    v1 = (v10, v11)
    sem_g = (sem_g0, sem_g1)
    sem_s = (sem_s0, sem_s1)

    g_d = [None] * NCHUNK
    s_d = [None] * NCHUNK
    for k in range(NCHUNK):
        b = k % 2
        if k >= 2:
            for d in s_d[k - 2]:
                d.wait()
        base = w * PER_W + k * CHUNK
        ia = pltpu.async_copy(src_hbm.at[pl.ds(base, CHUNK)], srcv[b], sem_i)
        ib = pltpu.async_copy(dst_hbm.at[pl.ds(base, CHUNK)], dstv[b], sem_i)
        ia.wait()
        ib.wait()
        g_d[k] = (pltpu.async_copy(t0_hbm.at[srcv[b]], v0[b], sem_g[b]),
                  pltpu.async_copy(t1_hbm.at[srcv[b]], v1[b], sem_g[b]))
        if k >= 1:
            p = (k - 1) % 2
            for d in g_d[k - 1]:
                d.wait()
            s_d[k - 1] = (
                pltpu.async_copy(v0[p], acc0.at[dstv[p]], sem_s[p], add=True),
                pltpu.async_copy(v1[p], acc1.at[dstv[p]], sem_s[p], add=True))
    b = (NCHUNK - 1) % 2
    for d in g_d[NCHUNK - 1]:
        d.wait()
    s_d[NCHUNK - 1] = (
        pltpu.async_copy(v0[b], acc0.at[dstv[b]], sem_s[b], add=True),
        pltpu.async_copy(v1[b], acc1.at[dstv[b]], sem_s[b], add=True))
    for d in s_d[NCHUNK - 2]:
        d.wait()
    for d in s_d[NCHUNK - 1]:
        d.wait()
    plsc.subcore_barrier()
    pltpu.sync_copy(acc0.at[sl], out_hbm.at[c, 0, sl])
    pltpu.sync_copy(acc1.at[sl], out_hbm.at[c, 1, sl])


def _l2_body(src_hbm, dst_hbm, t0_hbm, zeros_hbm, out_hbm,
             acc0, srcv0, srcv1, dstv0, dstv1, v00, v01,
             sem_i, sem_g0, sem_g1, sem_s0, sem_s1):
    c = lax.axis_index("c")
    s = lax.axis_index("s")
    w = s * NC + c
    sl = pl.ds(s * SLICE, SLICE)
    pltpu.sync_copy(zeros_hbm, acc0.at[sl])
    plsc.subcore_barrier()

    srcv = (srcv0, srcv1)
    dstv = (dstv0, dstv1)
    v0 = (v00, v01)
    sem_g = (sem_g0, sem_g1)
    sem_s = (sem_s0, sem_s1)

    g_d = [None] * NCHUNK
    s_d = [None] * NCHUNK
    for k in range(NCHUNK):
        b = k % 2
        if k >= 2:
            s_d[k - 2].wait()
        base = w * PER_W + k * CHUNK
        ia = pltpu.async_copy(src_hbm.at[pl.ds(base, CHUNK)], srcv[b], sem_i)
        ib = pltpu.async_copy(dst_hbm.at[pl.ds(base, CHUNK)], dstv[b], sem_i)
        ia.wait()
        ib.wait()
        g_d[k] = pltpu.async_copy(t0_hbm.at[srcv[b]], v0[b], sem_g[b])
        if k >= 1:
            p = (k - 1) % 2
            g_d[k - 1].wait()
            s_d[k - 1] = pltpu.async_copy(v0[p], acc0.at[dstv[p]], sem_s[p], add=True)
    b = (NCHUNK - 1) % 2
    g_d[NCHUNK - 1].wait()
    s_d[NCHUNK - 1] = pltpu.async_copy(v0[b], acc0.at[dstv[b]], sem_s[b], add=True)
    s_d[NCHUNK - 2].wait()
    s_d[NCHUNK - 1].wait()
    plsc.subcore_barrier()
    pltpu.sync_copy(acc0.at[sl], out_hbm.at[c, sl])


_deg_call = pl.kernel(
    _deg_body,
    out_type=jax.ShapeDtypeStruct((NC, NPAD), _f32),
    mesh=_mesh(),
    scratch_types=[
        pltpu.VMEM_SHARED((NPAD,), _f32),
        pltpu.VMEM((CHUNK,), jnp.int32),
        pltpu.VMEM((CHUNK,), jnp.int32),
        pltpu.VMEM((CHUNK,), _f32),
        pltpu.SemaphoreType.DMA,
        pltpu.SemaphoreType.DMA,
        pltpu.SemaphoreType.DMA,
        pltpu.SemaphoreType.DMA,
    ],
)

_l1_call = pl.kernel(
    _l1_body,
    out_type=jax.ShapeDtypeStruct((NC, 2, NPAD), _f32),
    mesh=_mesh(),
    scratch_types=[
        pltpu.VMEM_SHARED((NPAD,), _f32),
        pltpu.VMEM_SHARED((NPAD,), _f32),
        pltpu.VMEM((CHUNK,), jnp.int32),
        pltpu.VMEM((CHUNK,), jnp.int32),
        pltpu.VMEM((CHUNK,), jnp.int32),
        pltpu.VMEM((CHUNK,), jnp.int32),
        pltpu.VMEM((CHUNK,), _f32),
        pltpu.VMEM((CHUNK,), _f32),
        pltpu.VMEM((CHUNK,), _f32),
        pltpu.VMEM((CHUNK,), _f32),
        pltpu.SemaphoreType.DMA,
        pltpu.SemaphoreType.DMA,
        pltpu.SemaphoreType.DMA,
        pltpu.SemaphoreType.DMA,
        pltpu.SemaphoreType.DMA,
    ],
)

_l2_call = pl.kernel(
    _l2_body,
    out_type=jax.ShapeDtypeStruct((NC, NPAD), _f32),
    mesh=_mesh(),
    scratch_types=[
        pltpu.VMEM_SHARED((NPAD,), _f32),
        pltpu.VMEM((CHUNK,), jnp.int32),
        pltpu.VMEM((CHUNK,), jnp.int32),
        pltpu.VMEM((CHUNK,), jnp.int32),
        pltpu.VMEM((CHUNK,), jnp.int32),
        pltpu.VMEM((CHUNK,), _f32),
        pltpu.VMEM((CHUNK,), _f32),
        pltpu.SemaphoreType.DMA,
        pltpu.SemaphoreType.DMA,
        pltpu.SemaphoreType.DMA,
        pltpu.SemaphoreType.DMA,
        pltpu.SemaphoreType.DMA,
    ],
)


def _tc1_body(degp, x0, x1, dinv_o, y10_o, y11_o):
    deg = degp[0:1, :] + degp[1:2, :] + 1.0
    dinv = lax.rsqrt(deg)
    dinv_o[...] = dinv
    y10_o[...] = dinv * x0[...]
    y11_o[...] = dinv * x1[...]


def _tc2_body(a1p, x0, x1, dinv, w1t, b1, w2t, z_o, y2_o):
    dv = dinv[...]
    d2 = dv * dv
    ap = a1p[0] + a1p[1]
    xx = jnp.concatenate([x0[...], x1[...]], axis=0)
    ax = dv * ap + d2 * xx
    h = jnp.dot(w1t[...], ax, preferred_element_type=_f32) + b1[...]
    h = jnp.maximum(h, 0.0)
    z = jnp.dot(w2t[...], h, preferred_element_type=_f32)
    z_o[...] = z
    y2_o[...] = dv * z


def _tc3_body(a2p, z, dinv, b2, out_o):
    dv = dinv[...]
    out_o[...] = dv * (a2p[0:1, :] + a2p[1:2, :] + dv * z[...]) + b2[...]


_tc1_call = pl.pallas_call(
    _tc1_body,
    out_shape=(
        jax.ShapeDtypeStruct((1, NPAD), _f32),
        jax.ShapeDtypeStruct((1, NPAD), _f32),
        jax.ShapeDtypeStruct((1, NPAD), _f32),
    ),
)

_tc2_call = pl.pallas_call(
    _tc2_body,
    out_shape=(
        jax.ShapeDtypeStruct((1, NPAD), _f32),
        jax.ShapeDtypeStruct((1, NPAD), _f32),
    ),
)

_tc3_call = pl.pallas_call(
    _tc3_body,
    out_shape=jax.ShapeDtypeStruct((1, NPAD), _f32),
)


def kernel(x, edge_index, W1, b1, W2, b2):
    src = edge_index[0].astype(jnp.int32)
    dst = edge_index[1].astype(jnp.int32)
    pad = NPAD - N_NODES
    x0 = jnp.pad(x[:, 0], (0, pad)).reshape(1, NPAD)
    x1 = jnp.pad(x[:, 1], (0, pad)).reshape(1, NPAD)
    zeros_h = jnp.zeros((SLICE,), _f32)
    ones_h = jnp.ones((CHUNK,), _f32)
    w1t = W1.T
    w2t = W2.T
    b1c = b1.reshape(64, 1)
    b2c = b2.reshape(1, 1)

    degp = _deg_call(dst, zeros_h, ones_h)
    dinv, y10, y11 = _tc1_call(degp, x0, x1)
    a1p = _l1_call(src, dst, y10.reshape(NPAD), y11.reshape(NPAD), zeros_h)
    z, y2 = _tc2_call(a1p, x0, x1, dinv, w1t, b1c, w2t)
    a2p = _l2_call(src, dst, y2.reshape(NPAD), zeros_h)
    out = _tc3_call(a2p, z, dinv, b2c)
    return out.reshape(NPAD)[:N_NODES]


# speedup vs baseline: 174.7893x; 1.2223x over previous
import jax
import jax.numpy as jnp
from jax import lax
from jax.experimental import pallas as pl
from jax.experimental.pallas import tpu as pltpu
from jax.experimental.pallas import tpu_sc as plsc

N_NODES = 100000
N_EDGES = 1600000
NPAD = 102400
NC, NS = 2, 16
NW = NC * NS
PER_W = N_EDGES // NW
CHUNK = 2000
NCHUNK = PER_W // CHUNK
SLICE = NPAD // NS

_f32 = jnp.float32


def _mesh():
    return plsc.VectorSubcoreMesh(
        core_axis_name="c", subcore_axis_name="s", num_cores=NC, num_subcores=NS
    )


def _deg_body(dst_hbm, zeros_hbm, ones_hbm, out_hbm, acc,
              dstv0, dstv1, onesv, sem_i0, sem_i1, sem_s0, sem_s1):
    c = lax.axis_index("c")
    s = lax.axis_index("s")
    w = s * NC + c
    sl = pl.ds(s * SLICE, SLICE)
    pltpu.sync_copy(zeros_hbm, acc.at[sl])
    pltpu.sync_copy(ones_hbm, onesv)
    plsc.subcore_barrier()

    dstv = (dstv0, dstv1)
    sem_i = (sem_i0, sem_i1)
    sem_s = (sem_s0, sem_s1)
    i_d = [None] * NCHUNK
    s_d = [None] * NCHUNK
    for k in range(NCHUNK):
        b = k % 2
        if k >= 2:
            s_d[k - 2].wait()
        base = w * PER_W + k * CHUNK
        i_d[k] = pltpu.async_copy(dst_hbm.at[pl.ds(base, CHUNK)], dstv[b], sem_i[b])
        i_d[k].wait()
        s_d[k] = pltpu.async_copy(onesv, acc.at[dstv[b]], sem_s[b], add=True)
    s_d[NCHUNK - 2].wait()
    s_d[NCHUNK - 1].wait()
    plsc.subcore_barrier()
    pltpu.sync_copy(acc.at[sl], out_hbm.at[c, sl])


def _l1_body(src_hbm, dst_hbm, t0_hbm, t1_hbm, zeros_hbm, out_hbm,
             tab0, tab1, acc0, acc1, srcv0, srcv1, dstv0, dstv1,
             v00, v01, v10, v11,
             sem_i, sem_g0, sem_g1, sem_s0, sem_s1):
    c = lax.axis_index("c")
    s = lax.axis_index("s")
    w = s * NC + c
    sl = pl.ds(s * SLICE, SLICE)
    pltpu.sync_copy(t0_hbm.at[sl], tab0.at[sl])
    pltpu.sync_copy(t1_hbm.at[sl], tab1.at[sl])
    pltpu.sync_copy(zeros_hbm, acc0.at[sl])
    pltpu.sync_copy(zeros_hbm, acc1.at[sl])
    plsc.subcore_barrier()

    srcv = (srcv0, srcv1)
    dstv = (dstv0, dstv1)
    v0 = (v00, v01)
    v1 = (v10, v11)
    sem_g = (sem_g0, sem_g1)
    sem_s = (sem_s0, sem_s1)

    g_d = [None] * NCHUNK
    s_d = [None] * NCHUNK
    for k in range(NCHUNK):
        b = k % 2
        if k >= 2:
            for d in s_d[k - 2]:
                d.wait()
        base = w * PER_W + k * CHUNK
        ia = pltpu.async_copy(src_hbm.at[pl.ds(base, CHUNK)], srcv[b], sem_i)
        ib = pltpu.async_copy(dst_hbm.at[pl.ds(base, CHUNK)], dstv[b], sem_i)
        ia.wait()
        ib.wait()
        g_d[k] = (pltpu.async_copy(tab0.at[srcv[b]], v0[b], sem_g[b]),
                  pltpu.async_copy(tab1.at[srcv[b]], v1[b], sem_g[b]))
        if k >= 1:
            p = (k - 1) % 2
            for d in g_d[k - 1]:
                d.wait()
            s_d[k - 1] = (
                pltpu.async_copy(v0[p], acc0.at[dstv[p]], sem_s[p], add=True),
                pltpu.async_copy(v1[p], acc1.at[dstv[p]], sem_s[p], add=True))
    b = (NCHUNK - 1) % 2
    for d in g_d[NCHUNK - 1]:
        d.wait()
    s_d[NCHUNK - 1] = (
        pltpu.async_copy(v0[b], acc0.at[dstv[b]], sem_s[b], add=True),
        pltpu.async_copy(v1[b], acc1.at[dstv[b]], sem_s[b], add=True))
    for d in s_d[NCHUNK - 2]:
        d.wait()
    for d in s_d[NCHUNK - 1]:
        d.wait()
    plsc.subcore_barrier()
    pltpu.sync_copy(acc0.at[sl], out_hbm.at[c, 0, sl])
    pltpu.sync_copy(acc1.at[sl], out_hbm.at[c, 1, sl])


def _l2_body(src_hbm, dst_hbm, t0_hbm, zeros_hbm, out_hbm,
             tab, acc0, srcv0, srcv1, dstv0, dstv1, v00, v01,
             sem_i, sem_g0, sem_g1, sem_s0, sem_s1):
    c = lax.axis_index("c")
    s = lax.axis_index("s")
    w = s * NC + c
    sl = pl.ds(s * SLICE, SLICE)
    pltpu.sync_copy(t0_hbm.at[sl], tab.at[sl])
    pltpu.sync_copy(zeros_hbm, acc0.at[sl])
    plsc.subcore_barrier()

    srcv = (srcv0, srcv1)
    dstv = (dstv0, dstv1)
    v0 = (v00, v01)
    sem_g = (sem_g0, sem_g1)
    sem_s = (sem_s0, sem_s1)

    g_d = [None] * NCHUNK
    s_d = [None] * NCHUNK
    for k in range(NCHUNK):
        b = k % 2
        if k >= 2:
            s_d[k - 2].wait()
        base = w * PER_W + k * CHUNK
        ia = pltpu.async_copy(src_hbm.at[pl.ds(base, CHUNK)], srcv[b], sem_i)
        ib = pltpu.async_copy(dst_hbm.at[pl.ds(base, CHUNK)], dstv[b], sem_i)
        ia.wait()
        ib.wait()
        g_d[k] = pltpu.async_copy(tab.at[srcv[b]], v0[b], sem_g[b])
        if k >= 1:
            p = (k - 1) % 2
            g_d[k - 1].wait()
            s_d[k - 1] = pltpu.async_copy(v0[p], acc0.at[dstv[p]], sem_s[p], add=True)
    b = (NCHUNK - 1) % 2
    g_d[NCHUNK - 1].wait()
    s_d[NCHUNK - 1] = pltpu.async_copy(v0[b], acc0.at[dstv[b]], sem_s[b], add=True)
    s_d[NCHUNK - 2].wait()
    s_d[NCHUNK - 1].wait()
    plsc.subcore_barrier()
    pltpu.sync_copy(acc0.at[sl], out_hbm.at[c, sl])


_deg_call = pl.kernel(
    _deg_body,
    out_type=jax.ShapeDtypeStruct((NC, NPAD), _f32),
    mesh=_mesh(),
    scratch_types=[
        pltpu.VMEM_SHARED((NPAD,), _f32),
        pltpu.VMEM((CHUNK,), jnp.int32),
        pltpu.VMEM((CHUNK,), jnp.int32),
        pltpu.VMEM((CHUNK,), _f32),
        pltpu.SemaphoreType.DMA,
        pltpu.SemaphoreType.DMA,
        pltpu.SemaphoreType.DMA,
        pltpu.SemaphoreType.DMA,
    ],
)

_l1_call = pl.kernel(
    _l1_body,
    out_type=jax.ShapeDtypeStruct((NC, 2, NPAD), _f32),
    mesh=_mesh(),
    scratch_types=[
        pltpu.VMEM_SHARED((NPAD,), _f32),
        pltpu.VMEM_SHARED((NPAD,), _f32),
        pltpu.VMEM_SHARED((NPAD,), _f32),
        pltpu.VMEM_SHARED((NPAD,), _f32),
        pltpu.VMEM((CHUNK,), jnp.int32),
        pltpu.VMEM((CHUNK,), jnp.int32),
        pltpu.VMEM((CHUNK,), jnp.int32),
        pltpu.VMEM((CHUNK,), jnp.int32),
        pltpu.VMEM((CHUNK,), _f32),
        pltpu.VMEM((CHUNK,), _f32),
        pltpu.VMEM((CHUNK,), _f32),
        pltpu.VMEM((CHUNK,), _f32),
        pltpu.SemaphoreType.DMA,
        pltpu.SemaphoreType.DMA,
        pltpu.SemaphoreType.DMA,
        pltpu.SemaphoreType.DMA,
        pltpu.SemaphoreType.DMA,
    ],
)

_l2_call = pl.kernel(
    _l2_body,
    out_type=jax.ShapeDtypeStruct((NC, NPAD), _f32),
    mesh=_mesh(),
    scratch_types=[
        pltpu.VMEM_SHARED((NPAD,), _f32),
        pltpu.VMEM_SHARED((NPAD,), _f32),
        pltpu.VMEM((CHUNK,), jnp.int32),
        pltpu.VMEM((CHUNK,), jnp.int32),
        pltpu.VMEM((CHUNK,), jnp.int32),
        pltpu.VMEM((CHUNK,), jnp.int32),
        pltpu.VMEM((CHUNK,), _f32),
        pltpu.VMEM((CHUNK,), _f32),
        pltpu.SemaphoreType.DMA,
        pltpu.SemaphoreType.DMA,
        pltpu.SemaphoreType.DMA,
        pltpu.SemaphoreType.DMA,
        pltpu.SemaphoreType.DMA,
    ],
)


def _tc1_body(degp, x0, x1, dinv_o, y10_o, y11_o):
    deg = degp[0:1, :] + degp[1:2, :] + 1.0
    dinv = lax.rsqrt(deg)
    dinv_o[...] = dinv
    y10_o[...] = dinv * x0[...]
    y11_o[...] = dinv * x1[...]


def _tc2_body(a1p, x0, x1, dinv, w1t, b1, w2t, z_o, y2_o):
    dv = dinv[...]
    d2 = dv * dv
    ap = a1p[0] + a1p[1]
    xx = jnp.concatenate([x0[...], x1[...]], axis=0)
    ax = dv * ap + d2 * xx
    h = jnp.dot(w1t[...], ax, preferred_element_type=_f32) + b1[...]
    h = jnp.maximum(h, 0.0)
    z = jnp.dot(w2t[...], h, preferred_element_type=_f32)
    z_o[...] = z
    y2_o[...] = dv * z


def _tc3_body(a2p, z, dinv, b2, out_o):
    dv = dinv[...]
    out_o[...] = dv * (a2p[0:1, :] + a2p[1:2, :] + dv * z[...]) + b2[...]


_tc1_call = pl.pallas_call(
    _tc1_body,
    out_shape=(
        jax.ShapeDtypeStruct((1, NPAD), _f32),
        jax.ShapeDtypeStruct((1, NPAD), _f32),
        jax.ShapeDtypeStruct((1, NPAD), _f32),
    ),
)

_tc2_call = pl.pallas_call(
    _tc2_body,
    out_shape=(
        jax.ShapeDtypeStruct((1, NPAD), _f32),
        jax.ShapeDtypeStruct((1, NPAD), _f32),
    ),
)

_tc3_call = pl.pallas_call(
    _tc3_body,
    out_shape=jax.ShapeDtypeStruct((1, NPAD), _f32),
)


def kernel(x, edge_index, W1, b1, W2, b2):
    src = edge_index[0].astype(jnp.int32)
    dst = edge_index[1].astype(jnp.int32)
    pad = NPAD - N_NODES
    x0 = jnp.pad(x[:, 0], (0, pad)).reshape(1, NPAD)
    x1 = jnp.pad(x[:, 1], (0, pad)).reshape(1, NPAD)
    zeros_h = jnp.zeros((SLICE,), _f32)
    ones_h = jnp.ones((CHUNK,), _f32)
    w1t = W1.T
    w2t = W2.T
    b1c = b1.reshape(64, 1)
    b2c = b2.reshape(1, 1)

    degp = _deg_call(dst, zeros_h, ones_h)
    dinv, y10, y11 = _tc1_call(degp, x0, x1)
    a1p = _l1_call(src, dst, y10.reshape(NPAD), y11.reshape(NPAD), zeros_h)
    z, y2 = _tc2_call(a1p, x0, x1, dinv, w1t, b1c, w2t)
    a2p = _l2_call(src, dst, y2.reshape(NPAD), zeros_h)
    out = _tc3_call(a2p, z, dinv, b2c)
    return out.reshape(NPAD)[:N_NODES]
